# baseline (speedup 1.0000x reference)
import jax
import jax.numpy as jnp
from jax.experimental import pallas as pl
from jax.experimental.pallas import tpu as pltpu

_BUCKETS = 16
_HT = 56


def _weights_kernel(wt_ref, hwts_ref, acat_ref):
    w = wt_ref[...]
    wmax = jnp.max(w)
    wmin = jnp.min(w)
    step = (wmax - wmin) / _BUCKETS
    hw = w
    for i in range(_BUCKETS):
        thr = (i + 1) * step + wmin
        hw = jnp.where(hw > thr, hwts_ref[0, i], hw)
    for dh in range(3):
        a = jnp.concatenate([w[3 * dh], w[3 * dh + 1], w[3 * dh + 2]], axis=1)
        ah = jnp.concatenate([hw[3 * dh], hw[3 * dh + 1], hw[3 * dh + 2]], axis=1)
        acat_ref[dh] = jnp.concatenate([a, ah], axis=0)


def _conv_kernel(acat_ref, bcat_ref, x_ref, out1_ref, out2_ref, xp_ref):
    n = pl.program_id(0)
    h = pl.program_id(1)
    cin, hp, wp = xp_ref.shape
    himg = hp - 4
    wimg = wp - 2
    npos = (_HT + 1) * wp

    @pl.when(jnp.logical_and(n == 0, h == 0))
    def _():
        xp_ref[...] = jnp.zeros_like(xp_ref)

    @pl.when(h == 0)
    def _():
        xp_ref[:, 1 : himg + 1, 1 : wimg + 1] = x_ref[0]

    base = h * _HT
    acc = None
    for dh in range(3):
        xw = xp_ref[:, pl.ds(base + dh, _HT + 1), :].reshape(cin, npos)
        parts = [xw[:, dw : dw + _HT * wp] for dw in range(3)]
        xk = jnp.concatenate(parts, axis=0)
        d = jnp.dot(acat_ref[dh], xk, preferred_element_type=jnp.float32)
        acc = d if acc is None else acc + d
    y = (acc + bcat_ref[...]).reshape(-1, _HT, wp)
    cout = out1_ref.shape[1]
    out1_ref[0] = y[:cout, :, :wimg]
    out2_ref[0] = y[cout:, :, :wimg]


def kernel(x, W, b, b2, hashed_weights):
    n, cin, himg, wimg = x.shape
    cout = W.shape[0]
    nb = himg // _HT
    wp = wimg + 2

    wt = W.transpose(2, 3, 0, 1).reshape(9, cout, cin)
    bcat = jnp.concatenate([b, b2]).reshape(2 * cout, 1)
    hwts = hashed_weights.reshape(1, _BUCKETS)

    acat = pl.pallas_call(
        _weights_kernel,
        out_shape=jax.ShapeDtypeStruct((3, 2 * cout, 3 * cin), jnp.float32),
        name="hash_weights",
    )(wt, hwts)

    out_sds = jax.ShapeDtypeStruct((n, cout, himg, wimg), jnp.float32)
    out1, out2 = pl.pallas_call(
        _conv_kernel,
        grid=(n, nb),
        in_specs=[
            pl.BlockSpec((3, 2 * cout, 3 * cin), lambda i, j: (0, 0, 0)),
            pl.BlockSpec((2 * cout, 1), lambda i, j: (0, 0)),
            pl.BlockSpec((1, cin, himg, wimg), lambda i, j: (i, 0, 0, 0)),
        ],
        out_specs=[
            pl.BlockSpec((1, cout, _HT, wimg), lambda i, j: (i, 0, j, 0)),
            pl.BlockSpec((1, cout, _HT, wimg), lambda i, j: (i, 0, j, 0)),
        ],
        out_shape=[out_sds, out_sds],
        scratch_shapes=[pltpu.VMEM((cin, himg + 4, wp), jnp.float32)],
        compiler_params=pltpu.CompilerParams(
            dimension_semantics=("parallel", "arbitrary"),
            vmem_limit_bytes=52 * 1024 * 1024,
        ),
        name="fused_hashed_conv",
    )(acat, bcat, x)
    return (out1, out2)


# speedup vs baseline: 1.2276x; 1.2276x over previous
import jax
import jax.numpy as jnp
from jax.experimental import pallas as pl
from jax.experimental.pallas import tpu as pltpu

_BUCKETS = 16
_HT = 56
_WL = 128


def _weights_kernel(wt_ref, hwts_ref, acat_ref):
    w = wt_ref[...]
    wmax = jnp.max(w)
    wmin = jnp.min(w)
    step = (wmax - wmin) / _BUCKETS
    hw = w
    for i in range(_BUCKETS):
        thr = (i + 1) * step + wmin
        hw = jnp.where(hw > thr, hwts_ref[0, i], hw)
    for dh in range(3):
        a = jnp.concatenate([w[3 * dh], w[3 * dh + 1], w[3 * dh + 2]], axis=1)
        ah = jnp.concatenate([hw[3 * dh], hw[3 * dh + 1], hw[3 * dh + 2]], axis=1)
        acat_ref[dh] = jnp.concatenate([a, ah], axis=0).astype(jnp.bfloat16)


def _conv_kernel(acat_ref, bcat_ref, x_ref, out1_ref, out2_ref, xp_ref):
    h = pl.program_id(1)
    cin, hp, wl = xp_ref.shape
    himg = hp - 4
    wimg = x_ref.shape[3]
    npos = _HT * wl

    @pl.when(h == 0)
    def _():
        xp_ref[...] = jnp.zeros_like(xp_ref)
        xp_ref[:, 1 : himg + 1, 1 : wimg + 1] = x_ref[0]

    base = h * _HT
    acc = None
    for dh in range(3):
        xw = xp_ref[:, pl.ds(base + dh, _HT + 1), :].reshape(cin, npos + wl)
        parts = [xw[:, dw : dw + npos] for dw in range(3)]
        xk = jnp.concatenate(parts, axis=0).astype(jnp.bfloat16)
        d = jnp.dot(acat_ref[dh], xk, preferred_element_type=jnp.float32)
        acc = d if acc is None else acc + d
    y = (acc + bcat_ref[...]).reshape(-1, _HT, wl)
    cout = out1_ref.shape[1]
    out1_ref[0] = y[:cout, :, :wimg]
    out2_ref[0] = y[cout:, :, :wimg]


def kernel(x, W, b, b2, hashed_weights):
    n, cin, himg, wimg = x.shape
    cout = W.shape[0]
    nb = himg // _HT

    wt = W.transpose(2, 3, 0, 1).reshape(9, cout, cin)
    bcat = jnp.concatenate([b, b2]).reshape(2 * cout, 1)
    hwts = hashed_weights.reshape(1, _BUCKETS)

    acat = pl.pallas_call(
        _weights_kernel,
        out_shape=jax.ShapeDtypeStruct((3, 2 * cout, 3 * cin), jnp.bfloat16),
        name="hash_weights",
    )(wt, hwts)

    out_sds = jax.ShapeDtypeStruct((n, cout, himg, wimg), jnp.float32)
    out1, out2 = pl.pallas_call(
        _conv_kernel,
        grid=(n, nb),
        in_specs=[
            pl.BlockSpec((3, 2 * cout, 3 * cin), lambda i, j: (0, 0, 0)),
            pl.BlockSpec((2 * cout, 1), lambda i, j: (0, 0)),
            pl.BlockSpec((1, cin, himg, wimg), lambda i, j: (i, 0, 0, 0)),
        ],
        out_specs=[
            pl.BlockSpec((1, cout, _HT, wimg), lambda i, j: (i, 0, j, 0)),
            pl.BlockSpec((1, cout, _HT, wimg), lambda i, j: (i, 0, j, 0)),
        ],
        out_shape=[out_sds, out_sds],
        scratch_shapes=[pltpu.VMEM((cin, himg + 4, _WL), jnp.float32)],
        compiler_params=pltpu.CompilerParams(
            dimension_semantics=("parallel", "arbitrary"),
            vmem_limit_bytes=52 * 1024 * 1024,
        ),
        name="fused_hashed_conv",
    )(acat, bcat, x)
    return (out1, out2)


# speedup vs baseline: 1.3715x; 1.1172x over previous
import jax
import jax.numpy as jnp
from jax.experimental import pallas as pl
from jax.experimental.pallas import tpu as pltpu

_BUCKETS = 16
_HT = 56
_WL = 128


def _weights_kernel(wt_ref, hwts_ref, acat_ref):
    w = wt_ref[...]
    wmax = jnp.max(w)
    wmin = jnp.min(w)
    step = (wmax - wmin) / _BUCKETS
    hw = w
    for i in range(_BUCKETS):
        thr = (i + 1) * step + wmin
        hw = jnp.where(hw > thr, hwts_ref[0, i], hw)
    cols = []
    for dh in range(3):
        a = jnp.concatenate([w[3 * dh], w[3 * dh + 1], w[3 * dh + 2]], axis=1)
        ah = jnp.concatenate([hw[3 * dh], hw[3 * dh + 1], hw[3 * dh + 2]], axis=1)
        cols.append(jnp.concatenate([a, ah], axis=0))
    acat_ref[...] = jnp.concatenate(cols, axis=1).astype(jnp.bfloat16)


def _conv_kernel(acat_ref, bcat_ref, x_ref, out1_ref, out2_ref, xp_ref):
    h = pl.program_id(1)
    cin, hp, wl = xp_ref.shape
    himg = hp - 5
    wimg = x_ref.shape[3]
    npos = _HT * wl

    @pl.when(h == 0)
    def _():
        xp_ref[...] = jnp.zeros_like(xp_ref)
        xp_ref[:, 1 : himg + 1, 1 : wimg + 1] = x_ref[0]

    base = h * _HT
    winx = xp_ref[:, pl.ds(base, _HT + 3), :].reshape(cin, npos + 3 * wl)
    shifted = [
        winx[:, dw : dw + npos + 2 * wl].astype(jnp.bfloat16) for dw in range(3)
    ]
    xk = jnp.concatenate(
        [shifted[dw][:, dh * wl : dh * wl + npos] for dh in range(3) for dw in range(3)],
        axis=0,
    )
    acc = jnp.dot(acat_ref[...], xk, preferred_element_type=jnp.float32)
    y = (acc + bcat_ref[...]).reshape(-1, _HT, wl)
    cout = out1_ref.shape[1]
    out1_ref[0] = y[:cout, :, :wimg]
    out2_ref[0] = y[cout:, :, :wimg]


def kernel(x, W, b, b2, hashed_weights):
    n, cin, himg, wimg = x.shape
    cout = W.shape[0]
    nb = himg // _HT

    wt = W.transpose(2, 3, 0, 1).reshape(9, cout, cin)
    bcat = jnp.concatenate([b, b2]).reshape(2 * cout, 1)
    hwts = hashed_weights.reshape(1, _BUCKETS)

    acat = pl.pallas_call(
        _weights_kernel,
        out_shape=jax.ShapeDtypeStruct((2 * cout, 9 * cin), jnp.bfloat16),
        name="hash_weights",
    )(wt, hwts)

    out_sds = jax.ShapeDtypeStruct((n, cout, himg, wimg), jnp.float32)
    out1, out2 = pl.pallas_call(
        _conv_kernel,
        grid=(n, nb),
        in_specs=[
            pl.BlockSpec((2 * cout, 9 * cin), lambda i, j: (0, 0)),
            pl.BlockSpec((2 * cout, 1), lambda i, j: (0, 0)),
            pl.BlockSpec((1, cin, himg, wimg), lambda i, j: (i, 0, 0, 0)),
        ],
        out_specs=[
            pl.BlockSpec((1, cout, _HT, wimg), lambda i, j: (i, 0, j, 0)),
            pl.BlockSpec((1, cout, _HT, wimg), lambda i, j: (i, 0, j, 0)),
        ],
        out_shape=[out_sds, out_sds],
        scratch_shapes=[pltpu.VMEM((cin, himg + 5, _WL), jnp.float32)],
        compiler_params=pltpu.CompilerParams(
            dimension_semantics=("parallel", "arbitrary"),
            vmem_limit_bytes=52 * 1024 * 1024,
        ),
        name="fused_hashed_conv",
    )(acat, bcat, x)
    return (out1, out2)


# speedup vs baseline: 1.4079x; 1.0266x over previous
import jax
import jax.numpy as jnp
from jax.experimental import pallas as pl
from jax.experimental.pallas import tpu as pltpu

_BUCKETS = 16
_HT = 56
_WL = 128


def _weights_kernel(wt_ref, hwts_ref, bcat_ref, acat_ref):
    w = wt_ref[...]
    wmax = jnp.max(w)
    wmin = jnp.min(w)
    step = (wmax - wmin) / _BUCKETS
    hw = w
    for i in range(_BUCKETS):
        thr = (i + 1) * step + wmin
        hw = jnp.where(hw > thr, hwts_ref[0, i], hw)
    cols = []
    for dh in range(3):
        a = jnp.concatenate([w[3 * dh], w[3 * dh + 1], w[3 * dh + 2]], axis=1)
        ah = jnp.concatenate([hw[3 * dh], hw[3 * dh + 1], hw[3 * dh + 2]], axis=1)
        cols.append(jnp.concatenate([a, ah], axis=0))
    cols.append(bcat_ref[...])
    acat_ref[...] = jnp.concatenate(cols, axis=1).astype(jnp.bfloat16)


def _conv_kernel(acat_ref, x_ref, out1_ref, out2_ref, xp_ref):
    n = pl.program_id(0)
    h = pl.program_id(1)
    cin, hp, wl = xp_ref.shape
    himg = hp - 5
    wimg = x_ref.shape[3]
    npos = _HT * wl

    @pl.when(jnp.logical_and(n == 0, h == 0))
    def _():
        xp_ref[...] = jnp.zeros_like(xp_ref)

    @pl.when(h == 0)
    def _():
        xp_ref[:, 1 : himg + 1, 1 : wimg + 1] = x_ref[0]

    base = h * _HT
    winx = xp_ref[:, pl.ds(base, _HT + 3), :].reshape(cin, npos + 3 * wl)
    shifted = [
        winx[:, dw : dw + npos + 2 * wl].astype(jnp.bfloat16) for dw in range(3)
    ]
    xk = jnp.concatenate(
        [shifted[dw][:, dh * wl : dh * wl + npos] for dh in range(3) for dw in range(3)]
        + [jnp.ones((1, npos), jnp.bfloat16)],
        axis=0,
    )
    acc = jnp.dot(acat_ref[...], xk, preferred_element_type=jnp.float32)
    y = acc.reshape(-1, _HT, wl)
    cout = out1_ref.shape[1]
    out1_ref[0] = y[:cout, :, :wimg]
    out2_ref[0] = y[cout:, :, :wimg]


def kernel(x, W, b, b2, hashed_weights):
    n, cin, himg, wimg = x.shape
    cout = W.shape[0]
    nb = himg // _HT

    wt = W.transpose(2, 3, 0, 1).reshape(9, cout, cin)
    bcat = jnp.concatenate([b, b2]).reshape(2 * cout, 1)
    hwts = hashed_weights.reshape(1, _BUCKETS)

    acat = pl.pallas_call(
        _weights_kernel,
        out_shape=jax.ShapeDtypeStruct((2 * cout, 9 * cin + 1), jnp.bfloat16),
        name="hash_weights",
    )(wt, hwts, bcat)

    out_sds = jax.ShapeDtypeStruct((n, cout, himg, wimg), jnp.float32)
    out1, out2 = pl.pallas_call(
        _conv_kernel,
        grid=(n, nb),
        in_specs=[
            pl.BlockSpec((2 * cout, 9 * cin + 1), lambda i, j: (0, 0)),
            pl.BlockSpec((1, cin, himg, wimg), lambda i, j: (i, 0, 0, 0)),
        ],
        out_specs=[
            pl.BlockSpec((1, cout, _HT, wimg), lambda i, j: (i, 0, j, 0)),
            pl.BlockSpec((1, cout, _HT, wimg), lambda i, j: (i, 0, j, 0)),
        ],
        out_shape=[out_sds, out_sds],
        scratch_shapes=[pltpu.VMEM((cin, himg + 5, _WL), jnp.float32)],
        compiler_params=pltpu.CompilerParams(
            dimension_semantics=("parallel", "arbitrary"),
            vmem_limit_bytes=52 * 1024 * 1024,
        ),
        name="fused_hashed_conv",
    )(acat, x)
    return (out1, out2)


# speedup vs baseline: 1.5587x; 1.1071x over previous
import jax
import jax.numpy as jnp
from jax.experimental import pallas as pl
from jax.experimental.pallas import tpu as pltpu

_BUCKETS = 16
_HT = 56
_WL = 128


def _weights_kernel(wt_ref, hwts_ref, bcat_ref, acat_ref):
    w = wt_ref[...]
    wmax = jnp.max(w)
    wmin = jnp.min(w)
    step = (wmax - wmin) / _BUCKETS
    hw = w
    for i in range(_BUCKETS):
        thr = (i + 1) * step + wmin
        hw = jnp.where(hw > thr, hwts_ref[0, i], hw)
    cols = []
    for dh in range(3):
        a = jnp.concatenate([w[3 * dh], w[3 * dh + 1], w[3 * dh + 2]], axis=1)
        ah = jnp.concatenate([hw[3 * dh], hw[3 * dh + 1], hw[3 * dh + 2]], axis=1)
        cols.append(jnp.concatenate([a, ah], axis=0))
    cols.append(bcat_ref[...])
    acat_ref[...] = jnp.concatenate(cols, axis=1).astype(jnp.bfloat16)


def _conv_kernel(acat_ref, x_ref, out1_ref, out2_ref, xp_ref):
    n = pl.program_id(0)
    h = pl.program_id(1)
    cin, hp, wl = xp_ref.shape
    himg = hp - 5
    wimg = x_ref.shape[3]
    npos = _HT * wl

    @pl.when(jnp.logical_and(n == 0, h == 0))
    def _():
        xp_ref[...] = jnp.zeros_like(xp_ref)

    @pl.when(h == 0)
    def _():
        xp_ref[:, 1 : himg + 1, 1 : wimg + 1] = x_ref[0]

    base = h * _HT
    winx = xp_ref[:, pl.ds(base, _HT + 3), :].reshape(cin, npos + 3 * wl)
    shifted = [
        winx[:, dw : dw + npos + 2 * wl].astype(jnp.bfloat16) for dw in range(3)
    ]
    xk = jnp.concatenate(
        [shifted[dw][:, dh * wl : dh * wl + npos] for dh in range(3) for dw in range(3)]
        + [jnp.ones((1, npos), jnp.bfloat16)],
        axis=0,
    )
    acc = jnp.dot(acat_ref[...], xk, preferred_element_type=jnp.float32)
    y = jnp.concatenate(
        [acc[:, r * wl : r * wl + wimg] for r in range(_HT)], axis=1
    )
    cout = out1_ref.shape[1]
    out1_ref[0] = y[:cout]
    out2_ref[0] = y[cout:]


def kernel(x, W, b, b2, hashed_weights):
    n, cin, himg, wimg = x.shape
    cout = W.shape[0]
    nb = himg // _HT

    wt = W.transpose(2, 3, 0, 1).reshape(9, cout, cin)
    bcat = jnp.concatenate([b, b2]).reshape(2 * cout, 1)
    hwts = hashed_weights.reshape(1, _BUCKETS)

    acat = pl.pallas_call(
        _weights_kernel,
        out_shape=jax.ShapeDtypeStruct((2 * cout, 9 * cin + 1), jnp.bfloat16),
        name="hash_weights",
    )(wt, hwts, bcat)

    out_sds = jax.ShapeDtypeStruct((n, cout, himg * wimg), jnp.float32)
    out1, out2 = pl.pallas_call(
        _conv_kernel,
        grid=(n, nb),
        in_specs=[
            pl.BlockSpec((2 * cout, 9 * cin + 1), lambda i, j: (0, 0)),
            pl.BlockSpec((1, cin, himg, wimg), lambda i, j: (i, 0, 0, 0)),
        ],
        out_specs=[
            pl.BlockSpec((1, cout, _HT * wimg), lambda i, j: (i, 0, j)),
            pl.BlockSpec((1, cout, _HT * wimg), lambda i, j: (i, 0, j)),
        ],
        out_shape=[out_sds, out_sds],
        scratch_shapes=[pltpu.VMEM((cin, himg + 5, _WL), jnp.float32)],
        compiler_params=pltpu.CompilerParams(
            dimension_semantics=("parallel", "arbitrary"),
            vmem_limit_bytes=52 * 1024 * 1024,
        ),
        name="fused_hashed_conv",
    )(acat, x)
    shape = (n, cout, himg, wimg)
    return (out1.reshape(shape), out2.reshape(shape))
